# baseline (speedup 1.0000x reference)
import functools

import jax
import jax.numpy as jnp
from jax import lax
from jax.experimental import pallas as pl
from jax.experimental.pallas import tpu as pltpu
import jax.experimental.pallas.tpu_sc as plsc

_B = 4096
_NCAT = 26
_NNUM = 26
_NTEXT = 50
_D = 128
_P = _NCAT + _NNUM + _NTEXT
_PP = 104
_W = 32
_CK = 128
_NBUF = 4

_CAT_CH = _B * _NCAT // (_W * _CK)
_NUM_CH = _B * _NNUM // (_W * _CK)
_TXT_CH = _B * _NTEXT // (_W * _CK)
_NUM_SLAB = _NUM_CH * _CK


def _sc_body(cat_tbl, text_tbl, cat_add, num_add, text_add, param, x_t,
             cgi, cai, coi, nai, noi, tgi, tai, toi,
             out3,
             cg_v, ca_v, co_v, na_v, no_v, tg_v, ta_v, to_v,
             cat_sh, num_sh, text_sh,
             param_v, x_v, buf, sem_a, sem_b, sem_c):
    cid = lax.axis_index("c")
    sid = lax.axis_index("s")
    w = sid * 2 + cid
    out = out3

    @pl.when(sid == 0)
    def _():
        pltpu.sync_copy(cat_add, cat_sh)
        pltpu.sync_copy(num_add, num_sh)
        pltpu.sync_copy(text_add, text_sh)
    plsc.subcore_barrier()

    pltpu.sync_copy(cgi.at[w], cg_v)
    pltpu.sync_copy(cai.at[w], ca_v)
    pltpu.sync_copy(coi.at[w], co_v)
    pltpu.sync_copy(nai.at[w], na_v)
    pltpu.sync_copy(noi.at[w], no_v)
    pltpu.sync_copy(tgi.at[w], tg_v)
    pltpu.sync_copy(tai.at[w], ta_v)
    pltpu.sync_copy(toi.at[w], to_v)
    pltpu.sync_copy(param, param_v)
    pltpu.sync_copy(x_t.at[w], x_v)

    def slot_buf(s):
        return buf.at[pl.ds(s * _CK, _CK)]

    def drain_tail(n_ch, oi_v):
        for k in range(_NBUF):
            cc = n_ch - _NBUF + k
            s = cc % _NBUF
            pltpu.make_async_copy(slot_buf(s), out.at[oi_v.at[cc]],
                                  sem_c.at[s]).wait()

    def pipe_gather(n_ch, tbl, add_tbl, gi_v, ai_v, oi_v):
        def body(i, carry):
            @pl.when(i < n_ch)
            def _():
                s = lax.rem(i, _NBUF)

                @pl.when(i >= _NBUF)
                def _():
                    pltpu.make_async_copy(slot_buf(s),
                                          out.at[oi_v.at[i - _NBUF]],
                                          sem_c.at[s]).wait()
                pltpu.async_copy(add_tbl.at[ai_v.at[i]], slot_buf(s),
                                 sem_a.at[s])

            @pl.when((i >= 1) & (i <= n_ch))
            def _():
                c = i - 1
                s = lax.rem(c, _NBUF)
                pltpu.make_async_copy(add_tbl.at[ai_v.at[c]], slot_buf(s),
                                      sem_a.at[s]).wait()
                pltpu.async_copy(tbl.at[gi_v.at[c]], slot_buf(s),
                                 sem_b.at[s], add=True)

            @pl.when(i >= 2)
            def _():
                c = i - 2
                s = lax.rem(c, _NBUF)
                pltpu.make_async_copy(tbl.at[gi_v.at[c]], slot_buf(s),
                                      sem_b.at[s]).wait()
                pltpu.async_copy(slot_buf(s), out.at[oi_v.at[c]],
                                 sem_c.at[s])
            return carry
        lax.fori_loop(0, n_ch + 2, body, 0)
        drain_tail(n_ch, oi_v)

    pipe_gather(_CAT_CH, cat_tbl, cat_sh, cg_v, ca_v, co_v)

    def num_body(i, carry):
        @pl.when(i < _NUM_CH)
        def _():
            s = lax.rem(i, _NBUF)

            @pl.when(i >= _NBUF)
            def _():
                pltpu.make_async_copy(slot_buf(s),
                                      out.at[no_v.at[i - _NBUF]],
                                      sem_c.at[s]).wait()
            pltpu.async_copy(num_sh.at[na_v.at[i]], slot_buf(s),
                             sem_a.at[s])

        @pl.when(i >= 1)
        def _():
            c = i - 1
            s = lax.rem(c, _NBUF)
            pltpu.make_async_copy(num_sh.at[na_v.at[c]], slot_buf(s),
                                  sem_a.at[s]).wait()
            jrow = (w * _NUM_SLAB + c * _CK) // _B
            pv = [param_v[jrow, pl.ds(t * 16, 16)] for t in range(8)]

            def row_body(q, inner):
                xv = x_v[pl.ds(c * _CK + q * 16, 16)]
                for r in range(16):
                    xs = xv[r]
                    row = s * _CK + q * 16 + r
                    for t in range(8):
                        buf[row, pl.ds(t * 16, 16)] = (
                            buf[row, pl.ds(t * 16, 16)] + xs * pv[t])
                return inner
            lax.fori_loop(0, _CK // 16, row_body, 0)
            pltpu.async_copy(slot_buf(s), out.at[no_v.at[c]], sem_c.at[s])
        return carry
    lax.fori_loop(0, _NUM_CH + 1, num_body, 0)
    drain_tail(_NUM_CH, no_v)

    pipe_gather(_TXT_CH, text_tbl, text_sh, tg_v, ta_v, to_v)


@functools.partial(
    pl.kernel,
    out_type=jax.ShapeDtypeStruct((_B * _P, _D), jnp.float32),
    mesh=plsc.VectorSubcoreMesh(core_axis_name="c", subcore_axis_name="s",
                                num_cores=2, num_subcores=16),
    scratch_types=[
        pltpu.VMEM((_CAT_CH, _CK), jnp.int32),
        pltpu.VMEM((_CAT_CH, _CK), jnp.int32),
        pltpu.VMEM((_CAT_CH, _CK), jnp.int32),
        pltpu.VMEM((_NUM_CH, _CK), jnp.int32),
        pltpu.VMEM((_NUM_CH, _CK), jnp.int32),
        pltpu.VMEM((_TXT_CH, _CK), jnp.int32),
        pltpu.VMEM((_TXT_CH, _CK), jnp.int32),
        pltpu.VMEM((_TXT_CH, _CK), jnp.int32),
        pltpu.VMEM_SHARED((_NCAT * 6, _D), jnp.float32),
        pltpu.VMEM_SHARED((_NNUM * 6, _D), jnp.float32),
        pltpu.VMEM_SHARED((_NTEXT * 2, _D), jnp.float32),
        pltpu.VMEM((_NNUM, _D), jnp.float32),
        pltpu.VMEM((_NUM_SLAB,), jnp.float32),
        pltpu.VMEM((_NBUF * _CK, _D), jnp.float32),
        pltpu.SemaphoreType.DMA((_NBUF,)),
        pltpu.SemaphoreType.DMA((_NBUF,)),
        pltpu.SemaphoreType.DMA((_NBUF,)),
    ],
)
def _sc_kernel(*args):
    _sc_body(*args)


def kernel(cat_input_ids, cat_null_ids, cat_token_type, num_input_ids,
           num_null_ids, num_token_type, text_input_ids, text_token_type,
           cat_table, cat_pos_emb, num_emb_param, num_pos_emb,
           text_table, tt_table, null_table, pe_table):
    i32 = jnp.int32
    cat_add = (cat_pos_emb[:, None, None, :] + tt_table[None, :, None, :]
               + null_table[None, None, :, :]).reshape(_NCAT * 6, _D)
    num_add = (num_pos_emb[:, None, None, :] + tt_table[None, :, None, :]
               + null_table[None, None, :, :]).reshape(_NNUM * 6, _D)
    text_add = (pe_table[:, None, :] + tt_table[None, :, :]).reshape(
        _NTEXT * 2, _D)

    b = jnp.arange(_B, dtype=i32)[:, None]
    jc = jnp.arange(_NCAT, dtype=i32)[None, :]
    jt = jnp.arange(_NTEXT, dtype=i32)[None, :]

    cgi = cat_input_ids.astype(i32).reshape(_W, _CAT_CH, _CK)
    cai = (jc * 6 + cat_token_type.astype(i32) * 3
           + cat_null_ids.astype(i32)).reshape(_W, _CAT_CH, _CK)
    coi = (b * _P + jc).reshape(_W, _CAT_CH, _CK)

    nai = (jc * 6 + num_token_type.astype(i32) * 3
           + num_null_ids.astype(i32)).T.reshape(_W, _NUM_CH, _CK)
    noi = (b * _P + _NCAT + jc).T.reshape(_W, _NUM_CH, _CK)
    x_t = num_input_ids.astype(jnp.float32).T.reshape(_W, _NUM_SLAB)

    tgi = text_input_ids.astype(i32).reshape(_W, _TXT_CH, _CK)
    tai = (jt * 2 + text_token_type.astype(i32)).reshape(_W, _TXT_CH, _CK)
    toi = (b * _P + _NNUM + _NCAT + jt).reshape(_W, _TXT_CH, _CK)

    out2 = _sc_kernel(cat_table, text_table, cat_add, num_add, text_add,
                      num_emb_param, x_t,
                      cgi, cai, coi, nai, noi, tgi, tai, toi)
    return out2.reshape(_B, _P, _D)


# speedup vs baseline: 2.0583x; 2.0583x over previous
import functools

import jax
import jax.numpy as jnp
from jax import lax
from jax.experimental import pallas as pl
from jax.experimental.pallas import tpu as pltpu
import jax.experimental.pallas.tpu_sc as plsc

_B = 4096
_NCAT = 26
_NNUM = 26
_NTEXT = 50
_D = 128
_P = _NCAT + _NNUM + _NTEXT
_W = 32
_NB = _B // _W
_NBUF = 4
_NADD = _NCAT * 6 + _NNUM * 6 + _NTEXT * 2


def _sc_body(cat_tbl, text_tbl, comb_add, param, x_t, adi, cgi, tgi,
             out,
             ad_v, cg_v, tg_v, x_v, param_v, add_sh, buf,
             sem_a, sem_b, sem_t, sem_c):
    cid = lax.axis_index("c")
    sid = lax.axis_index("s")
    w = sid * 2 + cid
    b0 = w * _NB

    @pl.when(sid == 0)
    def _():
        pltpu.sync_copy(comb_add, add_sh)
    plsc.subcore_barrier()

    pltpu.sync_copy(adi.at[w], ad_v)
    pltpu.sync_copy(cgi.at[w], cg_v)
    pltpu.sync_copy(tgi.at[w], tg_v)
    pltpu.sync_copy(x_t.at[w], x_v)
    pltpu.sync_copy(param, param_v)

    def slab(s):
        return buf.at[pl.ds(s * _P, _P)]

    def body(i, carry):
        @pl.when(i < _NB)
        def _():
            s = lax.rem(i, _NBUF)

            @pl.when(i >= _NBUF)
            def _():
                pltpu.make_async_copy(slab(s), out.at[b0 + i - _NBUF],
                                      sem_c.at[s]).wait()
            pltpu.async_copy(add_sh.at[ad_v.at[i]], slab(s), sem_a.at[s])

        @pl.when((i >= 1) & (i <= _NB))
        def _():
            c = i - 1
            s = lax.rem(c, _NBUF)
            pltpu.make_async_copy(add_sh.at[ad_v.at[c]], slab(s),
                                  sem_a.at[s]).wait()
            pltpu.async_copy(cat_tbl.at[cg_v.at[c]],
                             buf.at[pl.ds(s * _P, _NCAT)],
                             sem_b.at[s], add=True)
            pltpu.async_copy(text_tbl.at[tg_v.at[c]],
                             buf.at[pl.ds(s * _P + _NCAT + _NNUM, _NTEXT)],
                             sem_t.at[s], add=True)
            xv0 = x_v[c, pl.ds(0, 16)]
            xv1 = x_v[c, pl.ds(16, 16)]
            for j in range(_NNUM):
                xs = xv0[j] if j < 16 else xv1[j - 16]
                row = s * _P + _NCAT + j
                for t in range(8):
                    buf[row, pl.ds(t * 16, 16)] = (
                        buf[row, pl.ds(t * 16, 16)]
                        + xs * param_v[j, pl.ds(t * 16, 16)])

        @pl.when(i >= 2)
        def _():
            c = i - 2
            s = lax.rem(c, _NBUF)
            pltpu.make_async_copy(cat_tbl.at[cg_v.at[c]],
                                  buf.at[pl.ds(s * _P, _NCAT)],
                                  sem_b.at[s]).wait()
            pltpu.make_async_copy(text_tbl.at[tg_v.at[c]],
                                  buf.at[pl.ds(s * _P + _NCAT + _NNUM,
                                               _NTEXT)],
                                  sem_t.at[s]).wait()
            pltpu.async_copy(slab(s), out.at[b0 + c], sem_c.at[s])
        return carry

    lax.fori_loop(0, _NB + 2, body, 0)

    for k in range(_NBUF):
        c = _NB - _NBUF + k
        s = c % _NBUF
        pltpu.make_async_copy(slab(s), out.at[b0 + c], sem_c.at[s]).wait()


@functools.partial(
    pl.kernel,
    out_type=jax.ShapeDtypeStruct((_B, _P, _D), jnp.float32),
    mesh=plsc.VectorSubcoreMesh(core_axis_name="c", subcore_axis_name="s",
                                num_cores=2, num_subcores=16),
    scratch_types=[
        pltpu.VMEM((_NB, _P), jnp.int32),
        pltpu.VMEM((_NB, _NCAT), jnp.int32),
        pltpu.VMEM((_NB, _NTEXT), jnp.int32),
        pltpu.VMEM((_NB, 32), jnp.float32),
        pltpu.VMEM((_NNUM, _D), jnp.float32),
        pltpu.VMEM_SHARED((_NADD, _D), jnp.float32),
        pltpu.VMEM((_NBUF * _P, _D), jnp.float32),
        pltpu.SemaphoreType.DMA((_NBUF,)),
        pltpu.SemaphoreType.DMA((_NBUF,)),
        pltpu.SemaphoreType.DMA((_NBUF,)),
        pltpu.SemaphoreType.DMA((_NBUF,)),
    ],
    compiler_params=pltpu.CompilerParams(use_tc_tiling_on_sc=True),
)
def _sc_kernel(*args):
    _sc_body(*args)


def kernel(cat_input_ids, cat_null_ids, cat_token_type, num_input_ids,
           num_null_ids, num_token_type, text_input_ids, text_token_type,
           cat_table, cat_pos_emb, num_emb_param, num_pos_emb,
           text_table, tt_table, null_table, pe_table):
    i32 = jnp.int32
    cat_add = (cat_pos_emb[:, None, None, :] + tt_table[None, :, None, :]
               + null_table[None, None, :, :]).reshape(_NCAT * 6, _D)
    num_add = (num_pos_emb[:, None, None, :] + tt_table[None, :, None, :]
               + null_table[None, None, :, :]).reshape(_NNUM * 6, _D)
    text_add = (pe_table[:, None, :] + tt_table[None, :, :]).reshape(
        _NTEXT * 2, _D)
    comb_add = jnp.concatenate([cat_add, num_add, text_add], axis=0)

    jc = jnp.arange(_NCAT, dtype=i32)[None, :]
    jt = jnp.arange(_NTEXT, dtype=i32)[None, :]

    cai = jc * 6 + cat_token_type.astype(i32) * 3 + cat_null_ids.astype(i32)
    nai = (_NCAT * 6 + jc * 6 + num_token_type.astype(i32) * 3
           + num_null_ids.astype(i32))
    tai = (_NCAT * 6 + _NNUM * 6 + jt * 2 + text_token_type.astype(i32))
    adi = jnp.concatenate([cai, nai, tai], axis=1).reshape(_W, _NB, _P)

    cgi = cat_input_ids.astype(i32).reshape(_W, _NB, _NCAT)
    tgi = text_input_ids.astype(i32).reshape(_W, _NB, _NTEXT)
    x_t = jnp.pad(num_input_ids.astype(jnp.float32),
                  ((0, 0), (0, 32 - _NNUM))).reshape(_W, _NB, 32)

    return _sc_kernel(cat_table, text_table, comb_add, num_emb_param, x_t,
                      adi, cgi, tgi)
